# baseline (speedup 1.0000x reference)
import functools

import jax
import jax.numpy as jnp
from jax import lax
from jax.experimental import pallas as pl
from jax.experimental.pallas import tpu as pltpu
from jax.experimental.pallas import tpu_sc as plsc

_NC = 2
_NS = 16
_NW = _NC * _NS
_LANES = 16


@functools.lru_cache(maxsize=None)
def _build_gather(batch, hist, vocab, d):
    bc = batch // _NW
    mesh = plsc.VectorSubcoreMesh(core_axis_name="c", subcore_axis_name="s")

    @functools.partial(
        pl.kernel,
        mesh=mesh,
        out_type=jax.ShapeDtypeStruct((hist, d, batch), jnp.float32),
        scratch_types=[
            pltpu.VMEM((hist, bc), jnp.int32),
            pltpu.VMEM((2, bc), jnp.int32),
            pltpu.VMEM((2, bc, 2 * d + 1), jnp.float32),
            pltpu.VMEM((2, d, bc), jnp.float32),
            pltpu.SemaphoreType.DMA((2,)),
            pltpu.SemaphoreType.DMA((2,)),
        ],
        compiler_params=pltpu.CompilerParams(needs_layout_passes=False),
    )
    def gather_kernel(x_t, scr, out, xv, idxv, linev, obuf, gsem, osem):
        wid = lax.axis_index("s") * _NC + lax.axis_index("c")
        b0 = wid * bc
        pltpu.sync_copy(x_t.at[:, pl.ds(b0, bc)], xv)
        iota = lax.iota(jnp.int32, _LANES)

        def prep(h, b):
            for g in range(bc // _LANES):
                v = xv[h, pl.ds(g * _LANES, _LANES)]
                idxv[b, pl.ds(g * _LANES, _LANES)] = (
                    lax.shift_right_logical(v, 1))

        def start_gather(b):
            pltpu.async_copy(
                scr.at[idxv.at[b]], linev.at[b, :, pl.ds(0, 2 * d)],
                gsem.at[b])

        def wait_gather(b):
            pltpu.make_async_copy(
                scr.at[idxv.at[b]], linev.at[b, :, pl.ds(0, 2 * d)],
                gsem.at[b]).wait()

        def out_copy(h, b):
            return pltpu.make_async_copy(
                obuf.at[b], out.at[h, :, pl.ds(b0, bc)], osem.at[b])

        rows = [iota + g * _LANES for g in range(bc // _LANES)]

        def build(h, b):
            pars = tuple(
                lax.shift_left(
                    jnp.bitwise_and(xv[h, pl.ds(g * _LANES, _LANES)], 1), 6)
                for g in range(bc // _LANES))

            def body(dd, carry):
                for g in range(bc // _LANES):
                    vals = plsc.load_gather(
                        linev.at[b], [rows[g], carry[g] + dd])
                    obuf[b, dd, pl.ds(g * _LANES, _LANES)] = vals
                return carry
            lax.fori_loop(0, d, body, pars)

        prep(0, 0)
        start_gather(0)
        prep(1, 1)

        def pair(p, carry):
            for s in range(2):
                h = p * 2 + s
                b = s
                wait_gather(b)

                @pl.when(h + 1 < hist)
                def _():
                    start_gather(1 - b)

                build(h, b)

                @pl.when(h >= 2)
                def _():
                    out_copy(h - 2, b).wait()

                out_copy(h, b).start()

                @pl.when(h + 2 < hist)
                def _():
                    prep(h + 2, b)
            return carry

        lax.fori_loop(0, hist // 2, pair, 0)
        out_copy(hist - 2, 0).wait()
        out_copy(hist - 1, 1).wait()

    return gather_kernel


def kernel(x, embedding):
    batch, hist = x.shape
    vocab, d = embedding.shape
    x_t = x.T.astype(jnp.int32)
    scr = embedding.reshape(vocab // 2, 2 * d)
    out_t = _build_gather(batch, hist, vocab, d)(x_t, scr)
    return out_t.transpose(2, 0, 1)


# speedup vs baseline: 1.1400x; 1.1400x over previous
import functools

import jax
import jax.numpy as jnp
from jax import lax
from jax.experimental import pallas as pl
from jax.experimental.pallas import tpu as pltpu
from jax.experimental.pallas import tpu_sc as plsc

_NC = 2
_NS = 16
_NW = _NC * _NS
_LANES = 16


@functools.lru_cache(maxsize=None)
def _build_gather(batch, hist, vocab, d):
    bc = batch // _NW
    mesh = plsc.VectorSubcoreMesh(core_axis_name="c", subcore_axis_name="s")

    @functools.partial(
        pl.kernel,
        mesh=mesh,
        out_type=jax.ShapeDtypeStruct((hist, d, batch), jnp.float32),
        scratch_types=[
            pltpu.VMEM((hist, bc), jnp.int32),
            pltpu.VMEM((2, bc), jnp.int32),
            pltpu.VMEM((2, bc, 2 * d), jnp.float32),
            pltpu.VMEM((2, d, bc + 1), jnp.float32),
            pltpu.SemaphoreType.DMA((2,)),
            pltpu.SemaphoreType.DMA((2,)),
        ],
        compiler_params=pltpu.CompilerParams(needs_layout_passes=False),
    )
    def gather_kernel(x_t, scr, out, xv, idxv, linev, obuf, gsem, osem):
        wid = lax.axis_index("s") * _NC + lax.axis_index("c")
        b0 = wid * bc
        pltpu.sync_copy(x_t.at[:, pl.ds(b0, bc)], xv)
        iota = lax.iota(jnp.int32, _LANES)

        def prep(h, b):
            for g in range(bc // _LANES):
                v = xv[h, pl.ds(g * _LANES, _LANES)]
                idxv[b, pl.ds(g * _LANES, _LANES)] = (
                    lax.shift_right_logical(v, 1))

        def start_gather(b):
            pltpu.async_copy(scr.at[idxv.at[b]], linev.at[b], gsem.at[b])

        def wait_gather(b):
            pltpu.make_async_copy(
                scr.at[idxv.at[b]], linev.at[b], gsem.at[b]).wait()

        def out_copy(h, b):
            return pltpu.make_async_copy(
                obuf.at[b, :, pl.ds(0, bc)], out.at[h, :, pl.ds(b0, bc)],
                osem.at[b])

        rows = [iota + k * _LANES for k in range(d // _LANES)]

        def build(h, b):
            def body(g, carry):
                offs = lax.shift_left(
                    jnp.bitwise_and(xv[h, pl.ds(g * _LANES, _LANES)], 1), 6)
                for lane in range(_LANES):
                    bb = g * _LANES + lane
                    off = offs[lane]
                    srcrow = jnp.full((_LANES,), bb, jnp.int32)
                    for k in range(d // _LANES):
                        vals = plsc.load_gather(
                            linev.at[b], [srcrow, rows[k] + off])
                        plsc.store_scatter(
                            obuf.at[b], [rows[k], srcrow], vals)
                return carry
            lax.fori_loop(0, bc // _LANES, body, 0)

        prep(0, 0)
        start_gather(0)
        prep(1, 1)

        def pair(p, carry):
            for s in range(2):
                h = p * 2 + s
                b = s
                wait_gather(b)

                @pl.when(h + 1 < hist)
                def _():
                    start_gather(1 - b)

                build(h, b)

                @pl.when(h >= 2)
                def _():
                    out_copy(h - 2, b).wait()

                out_copy(h, b).start()

                @pl.when(h + 2 < hist)
                def _():
                    prep(h + 2, b)
            return carry

        lax.fori_loop(0, hist // 2, pair, 0)
        out_copy(hist - 2, 0).wait()
        out_copy(hist - 1, 1).wait()

    return gather_kernel


def kernel(x, embedding):
    batch, hist = x.shape
    vocab, d = embedding.shape
    x_t = x.T.astype(jnp.int32)
    scr = embedding.reshape(vocab // 2, 2 * d)
    out_t = _build_gather(batch, hist, vocab, d)(x_t, scr)
    return out_t.transpose(2, 0, 1)


# speedup vs baseline: 1.4720x; 1.2913x over previous
import functools

import jax
import jax.numpy as jnp
from jax import lax
from jax.experimental import pallas as pl
from jax.experimental.pallas import tpu as pltpu
from jax.experimental.pallas import tpu_sc as plsc

_NC = 2
_NS = 16
_NW = _NC * _NS
_NBUF = 4


@functools.lru_cache(maxsize=None)
def _build(batch, hist, vocab, d):
    rows_per_tile = batch // _NW
    ch0 = (hist // 2) // 8 * 8
    ch1 = hist - ch0
    n_ch = rows_per_tile * 2
    mesh = plsc.VectorSubcoreMesh(core_axis_name="c", subcore_axis_name="s")

    def chunk(i, par):
        return i // 2, par * ch0, ch1 if par else ch0

    @functools.partial(
        pl.kernel,
        mesh=mesh,
        out_type=jax.ShapeDtypeStruct((batch, hist, d), jnp.float32),
        scratch_types=[
            pltpu.VMEM((rows_per_tile, hist), jnp.int32),
            pltpu.VMEM((_NBUF, ch1, d), jnp.float32),
            pltpu.SemaphoreType.DMA((_NBUF,)),
            pltpu.SemaphoreType.DMA((_NBUF,)),
        ],
        compiler_params=pltpu.CompilerParams(use_tc_tiling_on_sc=False),
    )
    def gather_kernel(x_hbm, table_hbm, out_hbm, xv, rows_v, gsem, osem):
        wid = lax.axis_index("s") * _NC + lax.axis_index("c")
        b0 = wid * rows_per_tile
        pltpu.sync_copy(x_hbm.at[pl.ds(b0, rows_per_tile)], xv)

        def gather_copy(i, b, par):
            r, h0, ln = chunk(i, par)
            return pltpu.make_async_copy(
                table_hbm.at[xv.at[r, pl.ds(h0, ln)]],
                rows_v.at[b, pl.ds(0, ln)], gsem.at[b])

        def out_copy(i, b, par):
            r, h0, ln = chunk(i, par)
            return pltpu.make_async_copy(
                rows_v.at[b, pl.ds(0, ln)],
                out_hbm.at[b0 + r, pl.ds(h0, ln), :], osem.at[b])

        gather_copy(0, 0, 0).start()
        gather_copy(1, 1, 1).start()
        for i in range(2):
            gather_copy(i, i, i % 2).wait()
            out_copy(i, i, i % 2).start()
            gather_copy(i + 2, i + 2, i % 2).start()

        def block(p, carry):
            i0 = 2 + p * _NBUF
            for dlt in range(_NBUF):
                i = i0 + dlt
                b = (2 + dlt) % _NBUF
                b2 = dlt % _NBUF
                par = dlt % 2
                gather_copy(i, b, par).wait()
                out_copy(i, b, par).start()
                out_copy(i - 2, b2, par).wait()
                gather_copy(i + 2, b2, par).start()
            return carry

        lax.fori_loop(0, (n_ch - 4) // _NBUF, block, 0)

        for i in range(n_ch - 2, n_ch):
            gather_copy(i, i % _NBUF, i % 2).wait()
            out_copy(i, i % _NBUF, i % 2).start()
        for i in range(n_ch - 4, n_ch):
            out_copy(i, i % _NBUF, i % 2).wait()

    return gather_kernel


def kernel(x, embedding):
    batch, hist = x.shape
    vocab, d = embedding.shape
    return _build(batch, hist, vocab, d)(x.astype(jnp.int32), embedding)
